# Initial kernel scaffold; baseline (speedup 1.0000x reference)
import functools

import jax
import jax.numpy as jnp
from jax import lax
from jax.experimental import pallas as pl
from jax.experimental.pallas import tpu as pltpu
from jax.experimental.pallas import tpu_sc as plsc

D = 32
LANES = 16
NC, NS = 2, 16
NW = NC * NS

BATCH = 4096
LEN_CONTEXT = 200
B_TOTAL = BATCH * LEN_CONTEXT
B_PER_W = B_TOTAL // NW
CHUNK = 1024
SUB = 128
NSUB = CHUNK // SUB
CHUNK_R = CHUNK // SUB
N_CHUNKS = B_PER_W // CHUNK

_mesh = plsc.VectorSubcoreMesh(core_axis_name="c", subcore_axis_name="s")


def _tanh_plus(x, p1):
    e = jnp.exp(x + x)
    return p1 - 2.0 / (e + 1.0)


@functools.partial(
    pl.kernel,
    mesh=_mesh,
    out_type=jax.ShapeDtypeStruct((2 * B_TOTAL, D), jnp.float32),
    scratch_types=[
        pltpu.VMEM((CHUNK_R, SUB), jnp.int32),
        pltpu.VMEM((CHUNK_R, SUB), jnp.int32),
        pltpu.VMEM((CHUNK_R, SUB), jnp.int32),
        pltpu.VMEM((CHUNK, D), jnp.float32),
        pltpu.VMEM((CHUNK, D), jnp.float32),
        pltpu.VMEM((CHUNK, D), jnp.float32),
        pltpu.SemaphoreType.DMA,
    ],
)
def _embed_sc(state_hbm, action_hbm, lp_hbm, ws_hbm, wa_hbm, wp_hbm,
              out_hbm, idx_s, idx_a, idx_p, rows_s, rows_a, rows_p, gsem):
    wid = lax.axis_index("s") * NC + lax.axis_index("c")
    base_r = wid * (B_PER_W // SUB)

    def chunk_body(c, _):
        off_r = base_r + c * CHUNK_R
        off = off_r * SUB
        pltpu.sync_copy(state_hbm.at[pl.ds(off_r, CHUNK_R)], idx_s)
        pltpu.sync_copy(action_hbm.at[pl.ds(off_r, CHUNK_R)], idx_a)
        pltpu.sync_copy(lp_hbm.at[pl.ds(off_r, CHUNK_R)], idx_p)
        copies = []
        for j in range(NSUB):
            dst = pl.ds(j * SUB, SUB)
            copies.append(pltpu.async_copy(ws_hbm.at[idx_s.at[j]], rows_s.at[dst], gsem))
            copies.append(pltpu.async_copy(wa_hbm.at[idx_a.at[j]], rows_a.at[dst], gsem))
            copies.append(pltpu.async_copy(wp_hbm.at[idx_p.at[j]], rows_p.at[dst], gsem))
        for cp in copies:
            cp.wait()

        def row_body(r, _):
            for h in range(2):
                sl = pl.ds(h * LANES, LANES)
                p1 = rows_p[r, sl] + 1.0
                rows_s[r, sl] = _tanh_plus(rows_s[r, sl], p1)
                rows_a[r, sl] = _tanh_plus(rows_a[r, sl], p1)
            return 0

        lax.fori_loop(0, CHUNK, row_body, 0)

        pltpu.sync_copy(rows_s, out_hbm.at[pl.ds(off, CHUNK)])
        pltpu.sync_copy(rows_a, out_hbm.at[pl.ds(B_TOTAL + off, CHUNK)])
        return 0

    lax.fori_loop(0, N_CHUNKS, chunk_body, 0)


def kernel(state, action, local_position, W_state, W_action, W_pos):
    s = state.reshape(B_TOTAL // SUB, SUB).astype(jnp.int32)
    a = action.reshape(B_TOTAL // SUB, SUB).astype(jnp.int32)
    p = local_position.reshape(B_TOTAL // SUB, SUB).astype(jnp.int32)
    out = _embed_sc(s, a, p, W_state, W_action, W_pos)
    return out.reshape(2, BATCH, LEN_CONTEXT, D)


# speedup vs baseline: 1.5465x; 1.5465x over previous
import functools

import jax
import jax.numpy as jnp
from jax import lax
from jax.experimental import pallas as pl
from jax.experimental.pallas import tpu as pltpu
from jax.experimental.pallas import tpu_sc as plsc

D = 32
LANES = 16
NC, NS = 2, 16
NW = NC * NS

BATCH = 4096
LEN_CONTEXT = 200
B_TOTAL = BATCH * LEN_CONTEXT
B_PER_W = B_TOTAL // NW
CHUNK = 1024
SUB = 128
NSUB = CHUNK // SUB
CHUNK_R = CHUNK // SUB
N_CHUNKS = B_PER_W // CHUNK

_mesh = plsc.VectorSubcoreMesh(core_axis_name="c", subcore_axis_name="s")


def _tanh_plus(x, p1):
    e = jnp.exp(x + x)
    return p1 - 2.0 / (e + 1.0)


@functools.partial(
    pl.kernel,
    mesh=_mesh,
    compiler_params=pltpu.CompilerParams(use_tc_tiling_on_sc=False),
    out_type=jax.ShapeDtypeStruct((2 * B_TOTAL, D), jnp.float32),
    scratch_types=[
        pltpu.VMEM((CHUNK_R, SUB), jnp.int32),
        pltpu.VMEM((CHUNK_R, SUB), jnp.int32),
        pltpu.VMEM((CHUNK_R, SUB), jnp.int32),
        pltpu.VMEM((CHUNK, D), jnp.float32),
        pltpu.VMEM((CHUNK, D), jnp.float32),
        pltpu.VMEM((CHUNK, D), jnp.float32),
        pltpu.SemaphoreType.DMA,
    ],
)
def _embed_sc(state_hbm, action_hbm, lp_hbm, ws_hbm, wa_hbm, wp_hbm,
              out_hbm, idx_s, idx_a, idx_p, rows_s, rows_a, rows_p, gsem):
    wid = lax.axis_index("s") * NC + lax.axis_index("c")
    base_r = wid * (B_PER_W // SUB)

    def chunk_body(c, _):
        off_r = base_r + c * CHUNK_R
        off = off_r * SUB
        pltpu.sync_copy(state_hbm.at[pl.ds(off_r, CHUNK_R)], idx_s)
        pltpu.sync_copy(action_hbm.at[pl.ds(off_r, CHUNK_R)], idx_a)
        pltpu.sync_copy(lp_hbm.at[pl.ds(off_r, CHUNK_R)], idx_p)
        copies = []
        for j in range(NSUB):
            dst = pl.ds(j * SUB, SUB)
            copies.append(pltpu.async_copy(ws_hbm.at[idx_s.at[j]], rows_s.at[dst], gsem))
            copies.append(pltpu.async_copy(wa_hbm.at[idx_a.at[j]], rows_a.at[dst], gsem))
            copies.append(pltpu.async_copy(wp_hbm.at[idx_p.at[j]], rows_p.at[dst], gsem))
        for cp in copies:
            cp.wait()

        def row_body(r, _):
            for h in range(2):
                sl = pl.ds(h * LANES, LANES)
                p1 = rows_p[r, sl] + 1.0
                rows_s[r, sl] = _tanh_plus(rows_s[r, sl], p1)
                rows_a[r, sl] = _tanh_plus(rows_a[r, sl], p1)
            return 0

        lax.fori_loop(0, CHUNK, row_body, 0)

        pltpu.sync_copy(rows_s, out_hbm.at[pl.ds(off, CHUNK)])
        pltpu.sync_copy(rows_a, out_hbm.at[pl.ds(B_TOTAL + off, CHUNK)])
        return 0

    lax.fori_loop(0, N_CHUNKS, chunk_body, 0)


def kernel(state, action, local_position, W_state, W_action, W_pos):
    s = state.reshape(B_TOTAL // SUB, SUB).astype(jnp.int32)
    a = action.reshape(B_TOTAL // SUB, SUB).astype(jnp.int32)
    p = local_position.reshape(B_TOTAL // SUB, SUB).astype(jnp.int32)
    out = _embed_sc(s, a, p, W_state, W_action, W_pos)
    return out.reshape(2, BATCH, LEN_CONTEXT, D)
